# baseline (speedup 1.0000x reference)
import functools

import jax
import jax.numpy as jnp
from jax import lax
from jax.experimental import pallas as pl
from jax.experimental.pallas import tpu as pltpu
from jax.experimental.pallas import tpu_sc as plsc

N_NODES = 10000
D = 128

NC = 2
NS = 16
NW = NC * NS
CHUNK = 128
SCH = 32
NB = 8
NCH = 80
NH = 5
ROWS_PER_SUB = 632
NPAD = NS * ROWS_PER_SUB
DUMMY_ROW = N_NODES


def _sc_gather_scatter(feature, edges_v, dst2d, tail_src, tail_dst, zeros_hbm):
    mesh = plsc.VectorSubcoreMesh(core_axis_name="c", subcore_axis_name="s")
    npc = NCH // NH
    spc = npc * CHUNK // SCH

    @functools.partial(
        pl.kernel,
        out_type=jax.ShapeDtypeStruct((NC, NPAD, D), jnp.float32),
        mesh=mesh,
        scratch_types=[
            pltpu.VMEM((npc, CHUNK), jnp.int32),
            pltpu.VMEM((spc, SCH), jnp.int32),
            pltpu.VMEM((NB, SCH, D), jnp.float32),
            pltpu.VMEM_SHARED((NPAD, D), jnp.float32),
            pltpu.SemaphoreType.DMA((NB,)),
            pltpu.SemaphoreType.DMA((NB,)),
        ],
    )
    def k(feat_hbm, edges_hbm, dst2d_hbm, tsrc_hbm, tdst_hbm, z_hbm, out_hbm,
          src_v, dst_v, buf, acc, gsem, ssem):
        c = lax.axis_index("c")
        s = lax.axis_index("s")
        wid = c * NS + s
        pltpu.sync_copy(z_hbm, acc.at[pl.ds(s * ROWS_PER_SUB, ROWS_PER_SUB)])
        plsc.subcore_barrier()

        def _src_slice(j, b):
            return src_v.at[(j * SCH) // CHUNK, pl.ds((b % (CHUNK // SCH)) * SCH, SCH)]

        def start_gather(j, b):
            pltpu.async_copy(feat_hbm.at[_src_slice(j, b)], buf.at[b],
                             gsem.at[b])

        def wait_gather(j, b):
            pltpu.make_async_copy(
                feat_hbm.at[_src_slice(j, b)], buf.at[b], gsem.at[b]).wait()

        def start_scatter(j, b):
            pltpu.async_copy(buf.at[b], acc.at[dst_v.at[j]], ssem.at[b],
                             add=True, priority=1)

        def wait_scatter(j, b):
            pltpu.make_async_copy(
                buf.at[b], acc.at[dst_v.at[j]], ssem.at[b]).wait()

        for h in range(NH):
            @pl.when(wid < NW - 1)
            def _():
                base = wid * NCH + h * npc
                pltpu.sync_copy(edges_hbm.at[0, pl.ds(base, npc)], src_v)
                pltpu.sync_copy(dst2d_hbm.at[pl.ds(base * (CHUNK // SCH), spc)], dst_v)

            @pl.when(wid == NW - 1)
            def _():
                pltpu.sync_copy(tsrc_hbm.at[pl.ds(h * npc, npc)], src_v)
                pltpu.sync_copy(tdst_hbm.at[pl.ds(h * spc, spc)], dst_v)

            for b in range(NB):
                start_gather(b, b)

            @pl.loop(0, spc // NB - 1)
            def _(m):
                base = m * NB
                for b in range(NB):
                    wait_gather(base + b, b)
                    start_scatter(base + b, b)
                for b in range(NB):
                    wait_scatter(base + b, b)
                    start_gather(base + NB + b, b)

            last = spc - NB
            for b in range(NB):
                wait_gather(last + b, b)
                start_scatter(last + b, b)
            for b in range(NB):
                wait_scatter(last + b, b)

        plsc.subcore_barrier()
        pltpu.sync_copy(
            acc.at[pl.ds(s * ROWS_PER_SUB, ROWS_PER_SUB)],
            out_hbm.at[c, pl.ds(s * ROWS_PER_SUB, ROWS_PER_SUB)],
        )

    return k(feature, edges_v, dst2d, tail_src, tail_dst, zeros_hbm)


def _tc_linear(h_parts, W, b2):
    blk = 1000

    def body(h_ref, w_ref, b_ref, o_ref):
        x = h_ref[0] + h_ref[1]
        o_ref[...] = lax.dot_general(
            x, w_ref[...], (((1,), (1,)), ((), ())),
            preferred_element_type=jnp.float32,
        ) + b_ref[...]

    return pl.pallas_call(
        body,
        out_shape=jax.ShapeDtypeStruct((N_NODES, D), jnp.float32),
        grid=(N_NODES // blk,),
        in_specs=[
            pl.BlockSpec((NC, blk, D), lambda i: (0, i, 0)),
            pl.BlockSpec((D, D), lambda i: (0, 0)),
            pl.BlockSpec((1, D), lambda i: (0, 0)),
        ],
        out_specs=pl.BlockSpec((blk, D), lambda i: (i, 0)),
    )(h_parts, W, b2)


def kernel(feature, edge_index, W, b):
    E = edge_index.shape[1]
    n_chunks = E // CHUNK
    assert n_chunks * CHUNK == E
    edges_v = edge_index.astype(jnp.int32).reshape(2, n_chunks, CHUNK)

    dst2d = edge_index[1].astype(jnp.int32).reshape(E // SCH, SCH)
    tail_e = (n_chunks - (NW - 1) * NCH) * CHUNK
    pad_e = NCH * CHUNK - tail_e
    pad_src = jnp.arange(pad_e, dtype=jnp.int32) % N_NODES
    pad_dst = DUMMY_ROW + jnp.arange(pad_e, dtype=jnp.int32) % (NPAD - DUMMY_ROW)
    tail_src = jnp.concatenate(
        [edges_v[0, (NW - 1) * NCH:].reshape(-1), pad_src]).reshape(NCH, CHUNK)
    tail_dst = jnp.concatenate(
        [edges_v[1, (NW - 1) * NCH:].reshape(-1), pad_dst]).reshape(
            NCH * CHUNK // SCH, SCH)
    zeros_hbm = jnp.zeros((ROWS_PER_SUB, D), jnp.float32)

    h_parts = _sc_gather_scatter(feature, edges_v, dst2d, tail_src, tail_dst,
                                 zeros_hbm)
    return _tc_linear(h_parts, W, b.reshape(1, D))


# speedup vs baseline: 1.0583x; 1.0583x over previous
import functools

import jax
import jax.numpy as jnp
from jax import lax
from jax.experimental import pallas as pl
from jax.experimental.pallas import tpu as pltpu
from jax.experimental.pallas import tpu_sc as plsc

N_NODES = 10000
D = 128

NC = 2
NS = 16
NW = NC * NS
CHUNK = 128
SCH = 64
NB = 4
NCH = 80
NH = 2
ROWS_PER_SUB = 632
NPAD = NS * ROWS_PER_SUB
DUMMY_ROW = N_NODES


def _sc_gather_scatter(feature, edges_v, dst2d, tail_src, tail_dst, zeros_hbm):
    mesh = plsc.VectorSubcoreMesh(core_axis_name="c", subcore_axis_name="s")
    npc = NCH // NH
    spc = npc * CHUNK // SCH

    @functools.partial(
        pl.kernel,
        out_type=jax.ShapeDtypeStruct((NC, NPAD, D), jnp.float32),
        mesh=mesh,
        scratch_types=[
            pltpu.VMEM((npc, CHUNK), jnp.int32),
            pltpu.VMEM((spc, SCH), jnp.int32),
            pltpu.VMEM((NB, SCH, D), jnp.float32),
            pltpu.VMEM_SHARED((NPAD, D), jnp.float32),
            pltpu.SemaphoreType.DMA((NB,)),
            pltpu.SemaphoreType.DMA((NB,)),
        ],
    )
    def k(feat_hbm, edges_hbm, dst2d_hbm, tsrc_hbm, tdst_hbm, z_hbm, out_hbm,
          src_v, dst_v, buf, acc, gsem, ssem):
        c = lax.axis_index("c")
        s = lax.axis_index("s")
        wid = c * NS + s
        pltpu.sync_copy(z_hbm, acc.at[pl.ds(s * ROWS_PER_SUB, ROWS_PER_SUB)])
        plsc.subcore_barrier()

        def _src_slice(j, b):
            return src_v.at[(j * SCH) // CHUNK, pl.ds((b % 2) * SCH, SCH)]

        def start_gather(j, b):
            pltpu.async_copy(feat_hbm.at[_src_slice(j, b)], buf.at[b],
                             gsem.at[b])

        def wait_gather(j, b):
            pltpu.make_async_copy(
                feat_hbm.at[_src_slice(j, b)], buf.at[b], gsem.at[b]).wait()

        def start_scatter(j, b):
            pltpu.async_copy(buf.at[b], acc.at[dst_v.at[j]], ssem.at[b],
                             add=True, priority=1)

        def wait_scatter(j, b):
            pltpu.make_async_copy(
                buf.at[b], acc.at[dst_v.at[j]], ssem.at[b]).wait()

        for h in range(NH):
            @pl.when(wid < NW - 1)
            def _():
                base = wid * NCH + h * npc
                pltpu.sync_copy(edges_hbm.at[0, pl.ds(base, npc)], src_v)
                pltpu.sync_copy(dst2d_hbm.at[pl.ds(base * 2, spc)], dst_v)

            @pl.when(wid == NW - 1)
            def _():
                pltpu.sync_copy(tsrc_hbm.at[pl.ds(h * npc, npc)], src_v)
                pltpu.sync_copy(tdst_hbm.at[pl.ds(h * spc, spc)], dst_v)

            for b in range(NB):
                start_gather(b, b)

            @pl.loop(0, spc // NB - 1)
            def _(m):
                base = m * NB
                for b in range(NB):
                    wait_gather(base + b, b)
                    start_scatter(base + b, b)
                for b in range(NB):
                    wait_scatter(base + b, b)
                    start_gather(base + NB + b, b)

            last = spc - NB
            for b in range(NB):
                wait_gather(last + b, b)
                start_scatter(last + b, b)
            for b in range(NB):
                wait_scatter(last + b, b)

        plsc.subcore_barrier()
        pltpu.sync_copy(
            acc.at[pl.ds(s * ROWS_PER_SUB, ROWS_PER_SUB)],
            out_hbm.at[c, pl.ds(s * ROWS_PER_SUB, ROWS_PER_SUB)],
        )

    return k(feature, edges_v, dst2d, tail_src, tail_dst, zeros_hbm)


def _tc_linear(h_parts, W, b2):
    blk = 2000

    def body(h_ref, w_ref, b_ref, o_ref):
        x = h_ref[0] + h_ref[1]
        o_ref[...] = lax.dot_general(
            x, w_ref[...], (((1,), (1,)), ((), ())),
            preferred_element_type=jnp.float32,
        ) + b_ref[...]

    return pl.pallas_call(
        body,
        out_shape=jax.ShapeDtypeStruct((N_NODES, D), jnp.float32),
        grid=(N_NODES // blk,),
        in_specs=[
            pl.BlockSpec((NC, blk, D), lambda i: (0, i, 0)),
            pl.BlockSpec((D, D), lambda i: (0, 0)),
            pl.BlockSpec((1, D), lambda i: (0, 0)),
        ],
        out_specs=pl.BlockSpec((blk, D), lambda i: (i, 0)),
    )(h_parts, W, b2)


def kernel(feature, edge_index, W, b):
    E = edge_index.shape[1]
    n_chunks = E // CHUNK
    assert n_chunks * CHUNK == E
    edges_v = edge_index.astype(jnp.int32).reshape(2, n_chunks, CHUNK)

    dst2d = edge_index[1].astype(jnp.int32).reshape(E // SCH, SCH)
    tail_e = (n_chunks - (NW - 1) * NCH) * CHUNK
    pad_e = NCH * CHUNK - tail_e
    pad_src = jnp.arange(pad_e, dtype=jnp.int32) % N_NODES
    pad_dst = DUMMY_ROW + jnp.arange(pad_e, dtype=jnp.int32) % (NPAD - DUMMY_ROW)
    tail_src = jnp.concatenate(
        [edges_v[0, (NW - 1) * NCH:].reshape(-1), pad_src]).reshape(NCH, CHUNK)
    tail_dst = jnp.concatenate(
        [edges_v[1, (NW - 1) * NCH:].reshape(-1), pad_dst]).reshape(
            NCH * CHUNK // SCH, SCH)
    zeros_hbm = jnp.zeros((ROWS_PER_SUB, D), jnp.float32)

    h_parts = _sc_gather_scatter(feature, edges_v, dst2d, tail_src, tail_dst,
                                 zeros_hbm)
    return _tc_linear(h_parts, W, b.reshape(1, D))


# speedup vs baseline: 1.0699x; 1.0110x over previous
import functools

import jax
import jax.numpy as jnp
from jax import lax
from jax.experimental import pallas as pl
from jax.experimental.pallas import tpu as pltpu
from jax.experimental.pallas import tpu_sc as plsc

N_NODES = 10000
D = 128

NC = 2
NS = 16
NW = NC * NS
CHUNK = 128
SCH = 64
NB = 4
NCH = 80
NH = 5
ROWS_PER_SUB = 632
NPAD = NS * ROWS_PER_SUB
DUMMY_ROW = N_NODES


def _sc_gather_scatter(feature, edges_v, dst2d, tail_src, tail_dst, zeros_hbm):
    mesh = plsc.VectorSubcoreMesh(core_axis_name="c", subcore_axis_name="s")
    npc = NCH // NH
    spc = npc * CHUNK // SCH

    @functools.partial(
        pl.kernel,
        out_type=jax.ShapeDtypeStruct((NC, NPAD, D), jnp.float32),
        mesh=mesh,
        scratch_types=[
            pltpu.VMEM((2, npc, CHUNK), jnp.int32),
            pltpu.VMEM((2, spc, SCH), jnp.int32),
            pltpu.VMEM((NB, SCH, D), jnp.float32),
            pltpu.VMEM_SHARED((NPAD, D), jnp.float32),
            pltpu.SemaphoreType.DMA((NB,)),
            pltpu.SemaphoreType.DMA((NB,)),
            pltpu.SemaphoreType.DMA,
            pltpu.SemaphoreType.DMA,
        ],
    )
    def k(feat_hbm, edges_hbm, dst2d_hbm, tsrc_hbm, tdst_hbm, z_hbm, out_hbm,
          src_v, dst_v, buf, acc, gsem, ssem, pssem, pdsem):
        c = lax.axis_index("c")
        s = lax.axis_index("s")
        wid = c * NS + s

        def _src_slice(p, j, b):
            return src_v.at[p % 2, (j * SCH) // CHUNK,
                            pl.ds((b % (CHUNK // SCH)) * SCH, SCH)]

        def start_gather(p, j, b):
            pltpu.async_copy(feat_hbm.at[_src_slice(p, j, b)], buf.at[b],
                             gsem.at[b])

        def wait_gather(p, j, b):
            pltpu.make_async_copy(
                feat_hbm.at[_src_slice(p, j, b)], buf.at[b], gsem.at[b]).wait()

        def start_scatter(p, j, b):
            pltpu.async_copy(buf.at[b], acc.at[dst_v.at[p % 2, j]],
                             ssem.at[b], add=True, priority=1)

        def wait_scatter(p, j, b):
            pltpu.make_async_copy(
                buf.at[b], acc.at[dst_v.at[p % 2, j]], ssem.at[b]).wait()

        def _idx_descs(p):
            base = wid * NCH + p * npc
            return (
                (edges_hbm.at[0, pl.ds(base, npc)],
                 dst2d_hbm.at[pl.ds(base * (CHUNK // SCH), spc)]),
                (tsrc_hbm.at[pl.ds(p * npc, npc)],
                 tdst_hbm.at[pl.ds(p * spc, spc)]),
            )

        def stage_sync(p):
            main, tail = _idx_descs(p)

            @pl.when(wid < NW - 1)
            def _():
                pltpu.sync_copy(main[0], src_v.at[p % 2])
                pltpu.sync_copy(main[1], dst_v.at[p % 2])

            @pl.when(wid == NW - 1)
            def _():
                pltpu.sync_copy(tail[0], src_v.at[p % 2])
                pltpu.sync_copy(tail[1], dst_v.at[p % 2])

        def prefetch_start(p):
            main, tail = _idx_descs(p)

            @pl.when(wid < NW - 1)
            def _():
                pltpu.async_copy(main[0], src_v.at[p % 2], pssem)
                pltpu.async_copy(main[1], dst_v.at[p % 2], pdsem)

            @pl.when(wid == NW - 1)
            def _():
                pltpu.async_copy(tail[0], src_v.at[p % 2], pssem)
                pltpu.async_copy(tail[1], dst_v.at[p % 2], pdsem)

        def prefetch_wait(p):
            main, tail = _idx_descs(p)

            @pl.when(wid < NW - 1)
            def _():
                pltpu.make_async_copy(main[0], src_v.at[p % 2], pssem).wait()
                pltpu.make_async_copy(main[1], dst_v.at[p % 2], pdsem).wait()

            @pl.when(wid == NW - 1)
            def _():
                pltpu.make_async_copy(tail[0], src_v.at[p % 2], pssem).wait()
                pltpu.make_async_copy(tail[1], dst_v.at[p % 2], pdsem).wait()

        stage_sync(0)
        pltpu.sync_copy(z_hbm, acc.at[pl.ds(s * ROWS_PER_SUB, ROWS_PER_SUB)])
        plsc.subcore_barrier()
        for b in range(NB):
            start_gather(0, b, b)

        ngr = spc // NB
        for p in range(NH):
            if p + 1 < NH:
                prefetch_start(p + 1)

            @pl.loop(0, ngr - 1)
            def _(m):
                base = m * NB
                for b in range(NB):
                    wait_gather(p, base + b, b)
                    start_scatter(p, base + b, b)
                for b in range(NB):
                    wait_scatter(p, base + b, b)
                    start_gather(p, base + NB + b, b)

            last = spc - NB
            for b in range(NB):
                wait_gather(p, last + b, b)
                start_scatter(p, last + b, b)
            if p + 1 < NH:
                prefetch_wait(p + 1)
            for b in range(NB):
                wait_scatter(p, last + b, b)
                if p + 1 < NH:
                    start_gather(p + 1, b, b)

        plsc.subcore_barrier()
        pltpu.sync_copy(
            acc.at[pl.ds(s * ROWS_PER_SUB, ROWS_PER_SUB)],
            out_hbm.at[c, pl.ds(s * ROWS_PER_SUB, ROWS_PER_SUB)],
        )

    return k(feature, edges_v, dst2d, tail_src, tail_dst, zeros_hbm)


def _tc_linear(h_parts, W, b2):
    blk = 2000

    def body(h_ref, w_ref, b_ref, o_ref):
        x = h_ref[0] + h_ref[1]
        o_ref[...] = lax.dot_general(
            x, w_ref[...], (((1,), (1,)), ((), ())),
            preferred_element_type=jnp.float32,
        ) + b_ref[...]

    return pl.pallas_call(
        body,
        out_shape=jax.ShapeDtypeStruct((N_NODES, D), jnp.float32),
        grid=(N_NODES // blk,),
        in_specs=[
            pl.BlockSpec((NC, blk, D), lambda i: (0, i, 0)),
            pl.BlockSpec((D, D), lambda i: (0, 0)),
            pl.BlockSpec((1, D), lambda i: (0, 0)),
        ],
        out_specs=pl.BlockSpec((blk, D), lambda i: (i, 0)),
    )(h_parts, W, b2)


def kernel(feature, edge_index, W, b):
    E = edge_index.shape[1]
    n_chunks = E // CHUNK
    assert n_chunks * CHUNK == E
    edges_v = edge_index.astype(jnp.int32).reshape(2, n_chunks, CHUNK)

    dst2d = edge_index[1].astype(jnp.int32).reshape(E // SCH, SCH)
    tail_e = (n_chunks - (NW - 1) * NCH) * CHUNK
    pad_e = NCH * CHUNK - tail_e
    pad_src = jnp.arange(pad_e, dtype=jnp.int32) % N_NODES
    pad_dst = DUMMY_ROW + jnp.arange(pad_e, dtype=jnp.int32) % (NPAD - DUMMY_ROW)
    tail_src = jnp.concatenate(
        [edges_v[0, (NW - 1) * NCH:].reshape(-1), pad_src]).reshape(NCH, CHUNK)
    tail_dst = jnp.concatenate(
        [edges_v[1, (NW - 1) * NCH:].reshape(-1), pad_dst]).reshape(
            NCH * CHUNK // SCH, SCH)
    zeros_hbm = jnp.zeros((ROWS_PER_SUB, D), jnp.float32)

    h_parts = _sc_gather_scatter(feature, edges_v, dst2d, tail_src, tail_dst,
                                 zeros_hbm)
    return _tc_linear(h_parts, W, b.reshape(1, D))


# speedup vs baseline: 1.0841x; 1.0133x over previous
import functools

import jax
import jax.numpy as jnp
from jax import lax
from jax.experimental import pallas as pl
from jax.experimental.pallas import tpu as pltpu
from jax.experimental.pallas import tpu_sc as plsc

N_NODES = 10000
D = 128

NC = 2
NS = 16
NW = NC * NS
CHUNK = 128
SCH = 64
NB = 4
NCH = 80
NH = 5
ROWS_PER_SUB = 632
NPAD = NS * ROWS_PER_SUB
DUMMY_ROW = N_NODES


def _sc_gather_scatter(feature, edges_v, dst2d, tail_src, tail_dst, zeros_hbm):
    mesh = plsc.VectorSubcoreMesh(core_axis_name="c", subcore_axis_name="s")
    npc = NCH // NH
    spc = npc * CHUNK // SCH

    @functools.partial(
        pl.kernel,
        out_type=jax.ShapeDtypeStruct((NC, NPAD, D), jnp.float32),
        mesh=mesh,
        scratch_types=[
            pltpu.VMEM((2, npc, CHUNK), jnp.int32),
            pltpu.VMEM((2, spc, SCH), jnp.int32),
            pltpu.VMEM((NB, SCH, D), jnp.float32),
            pltpu.VMEM_SHARED((NPAD, D), jnp.float32),
            pltpu.SemaphoreType.DMA((NB,)),
            pltpu.SemaphoreType.DMA((NB,)),
            pltpu.SemaphoreType.DMA,
            pltpu.SemaphoreType.DMA,
            pltpu.SemaphoreType.DMA,
        ],
    )
    def k(feat_hbm, edges_hbm, dst2d_hbm, tsrc_hbm, tdst_hbm, z_hbm, out_hbm,
          src_v, dst_v, buf, acc, gsem, ssem, pssem, pdsem, zsem):
        c = lax.axis_index("c")
        s = lax.axis_index("s")
        wid = c * NS + s

        def _src_slice(p, j, b):
            return src_v.at[p % 2, (j * SCH) // CHUNK,
                            pl.ds((b % (CHUNK // SCH)) * SCH, SCH)]

        def start_gather(p, j, b):
            pltpu.async_copy(feat_hbm.at[_src_slice(p, j, b)], buf.at[b],
                             gsem.at[b])

        def wait_gather(p, j, b):
            pltpu.make_async_copy(
                feat_hbm.at[_src_slice(p, j, b)], buf.at[b], gsem.at[b]).wait()

        def start_scatter(p, j, b):
            pltpu.async_copy(buf.at[b], acc.at[dst_v.at[p % 2, j]],
                             ssem.at[b], add=True, priority=1)

        def wait_scatter(p, j, b):
            pltpu.make_async_copy(
                buf.at[b], acc.at[dst_v.at[p % 2, j]], ssem.at[b]).wait()

        def _idx_descs(p):
            base = wid * NCH + p * npc
            return (
                (edges_hbm.at[0, pl.ds(base, npc)],
                 dst2d_hbm.at[pl.ds(base * (CHUNK // SCH), spc)]),
                (tsrc_hbm.at[pl.ds(p * npc, npc)],
                 tdst_hbm.at[pl.ds(p * spc, spc)]),
            )

        def stage_sync(p):
            main, tail = _idx_descs(p)

            @pl.when(wid < NW - 1)
            def _():
                pltpu.sync_copy(main[0], src_v.at[p % 2])
                pltpu.sync_copy(main[1], dst_v.at[p % 2])

            @pl.when(wid == NW - 1)
            def _():
                pltpu.sync_copy(tail[0], src_v.at[p % 2])
                pltpu.sync_copy(tail[1], dst_v.at[p % 2])

        def prefetch_start(p):
            main, tail = _idx_descs(p)

            @pl.when(wid < NW - 1)
            def _():
                pltpu.async_copy(main[0], src_v.at[p % 2], pssem)
                pltpu.async_copy(main[1], dst_v.at[p % 2], pdsem)

            @pl.when(wid == NW - 1)
            def _():
                pltpu.async_copy(tail[0], src_v.at[p % 2], pssem)
                pltpu.async_copy(tail[1], dst_v.at[p % 2], pdsem)

        def prefetch_wait(p):
            main, tail = _idx_descs(p)

            @pl.when(wid < NW - 1)
            def _():
                pltpu.make_async_copy(main[0], src_v.at[p % 2], pssem).wait()
                pltpu.make_async_copy(main[1], dst_v.at[p % 2], pdsem).wait()

            @pl.when(wid == NW - 1)
            def _():
                pltpu.make_async_copy(tail[0], src_v.at[p % 2], pssem).wait()
                pltpu.make_async_copy(tail[1], dst_v.at[p % 2], pdsem).wait()

        zdst = acc.at[pl.ds(s * ROWS_PER_SUB, ROWS_PER_SUB)]
        pltpu.async_copy(z_hbm, zdst, zsem)
        stage_sync(0)
        for b in range(NB):
            start_gather(0, b, b)
        pltpu.make_async_copy(z_hbm, zdst, zsem).wait()
        plsc.subcore_barrier()

        ngr = spc // NB
        for p in range(NH):
            if p + 1 < NH:
                prefetch_start(p + 1)

            @pl.loop(0, ngr - 1)
            def _(m):
                base = m * NB
                for b in range(NB):
                    wait_gather(p, base + b, b)
                    start_scatter(p, base + b, b)
                for b in range(NB):
                    wait_scatter(p, base + b, b)
                    start_gather(p, base + NB + b, b)

            last = spc - NB
            for b in range(NB):
                wait_gather(p, last + b, b)
                start_scatter(p, last + b, b)
            if p + 1 < NH:
                prefetch_wait(p + 1)
            for b in range(NB):
                wait_scatter(p, last + b, b)
                if p + 1 < NH:
                    start_gather(p + 1, b, b)

        plsc.subcore_barrier()
        pltpu.sync_copy(
            acc.at[pl.ds(s * ROWS_PER_SUB, ROWS_PER_SUB)],
            out_hbm.at[c, pl.ds(s * ROWS_PER_SUB, ROWS_PER_SUB)],
        )

    return k(feature, edges_v, dst2d, tail_src, tail_dst, zeros_hbm)


def _tc_linear(h_parts, W, b2):
    blk = 2000

    def body(h_ref, w_ref, b_ref, o_ref):
        x = h_ref[0] + h_ref[1]
        o_ref[...] = lax.dot_general(
            x, w_ref[...], (((1,), (1,)), ((), ())),
            preferred_element_type=jnp.float32,
        ) + b_ref[...]

    return pl.pallas_call(
        body,
        out_shape=jax.ShapeDtypeStruct((N_NODES, D), jnp.float32),
        grid=(N_NODES // blk,),
        in_specs=[
            pl.BlockSpec((NC, blk, D), lambda i: (0, i, 0)),
            pl.BlockSpec((D, D), lambda i: (0, 0)),
            pl.BlockSpec((1, D), lambda i: (0, 0)),
        ],
        out_specs=pl.BlockSpec((blk, D), lambda i: (i, 0)),
    )(h_parts, W, b2)


def kernel(feature, edge_index, W, b):
    E = edge_index.shape[1]
    n_chunks = E // CHUNK
    assert n_chunks * CHUNK == E
    edges_v = edge_index.astype(jnp.int32).reshape(2, n_chunks, CHUNK)

    dst2d = edge_index[1].astype(jnp.int32).reshape(E // SCH, SCH)
    tail_e = (n_chunks - (NW - 1) * NCH) * CHUNK
    pad_e = NCH * CHUNK - tail_e
    pad_src = jnp.arange(pad_e, dtype=jnp.int32) % N_NODES
    pad_dst = DUMMY_ROW + jnp.arange(pad_e, dtype=jnp.int32) % (NPAD - DUMMY_ROW)
    tail_src = jnp.concatenate(
        [edges_v[0, (NW - 1) * NCH:].reshape(-1), pad_src]).reshape(NCH, CHUNK)
    tail_dst = jnp.concatenate(
        [edges_v[1, (NW - 1) * NCH:].reshape(-1), pad_dst]).reshape(
            NCH * CHUNK // SCH, SCH)
    zeros_hbm = jnp.zeros((ROWS_PER_SUB, D), jnp.float32)

    h_parts = _sc_gather_scatter(feature, edges_v, dst2d, tail_src, tail_dst,
                                 zeros_hbm)
    return _tc_linear(h_parts, W, b.reshape(1, D))
